# baseline (speedup 1.0000x reference)
import functools

import jax
import jax.numpy as jnp
from jax import lax
from jax.experimental import pallas as pl
from jax.experimental.pallas import tpu as pltpu
from jax.experimental.pallas import tpu_sc as plsc

N_NODES = 10000
D_IN = 128
D_HID = 128
D_OUT = 64
N_PAD = 10240
E = 320000
CHUNK = 128
NC, NS = 2, 16
NW = NC * NS
E_TILE = 10240
E_PAD = E_TILE * NW
NCHUNK = E_TILE // CHUNK
STRIPE = N_PAD // NS


def _mesh():
    return plsc.VectorSubcoreMesh(
        core_axis_name="c", subcore_axis_name="s", num_cores=NC, num_subcores=NS
    )


E_TILE2 = E_PAD // NS
NCHUNK2 = E_TILE2 // CHUNK


def _sc_deg_agg1(y_bits, src1d, dst2d, ones_blk, zeros16, zeros_stripe, dh):
    NBUF = 2
    LAG = 1
    K = 8
    WPR = dh // 32

    def body(y_hbm, src_hbm, dst_hbm, ones_hbm, z16_hbm, zacc_hbm, dinv_hbm, out_hbm,
             src_v, dst_v, ones_v, deg_v, rows_i, rows_f, dinv_v, deg_sh, acc_sh,
             *sems):
        gsems = sems[:NBUF]
        ssems = sems[NBUF : 2 * NBUF]
        dsems = sems[2 * NBUF : 3 * NBUF]
        degsem = sems[3 * NBUF]
        cid = lax.axis_index("c")
        sid = lax.axis_index("s")
        pltpu.sync_copy(z16_hbm, deg_sh.at[pl.ds(sid * STRIPE, STRIPE)])
        pltpu.sync_copy(zacc_hbm, acc_sh.at[pl.ds(sid * STRIPE, STRIPE)])
        pltpu.sync_copy(ones_hbm, ones_v)
        pltpu.sync_copy(src_hbm.at[pl.ds(sid * E_TILE2, E_TILE2)], src_v)
        pltpu.sync_copy(dst_hbm.at[pl.ds(sid * NCHUNK2, NCHUNK2)], dst_v)
        plsc.subcore_barrier()

        @pl.loop(0, NCHUNK2, step=K)
        def _(j0):
            for t in range(K):
                pltpu.async_copy(ones_v, deg_sh.at[dst_v.at[j0 + t]], degsem, add=True)
            for t in range(K):
                pltpu.make_async_copy(ones_v, deg_sh.at[dst_v.at[j0 + t]], degsem).wait()

        plsc.subcore_barrier()

        for blk in range(STRIPE // CHUNK):
            base = sid * STRIPE + blk * CHUNK
            pltpu.sync_copy(deg_sh.at[pl.ds(base, CHUNK)], deg_v)

            @pl.loop(0, CHUNK, unroll=4)
            def _(r):
                d = deg_v[r, pl.ds(0, 16)] + 1.0
                u = jnp.int32(0x5F3759DF) - lax.shift_right_logical(
                    plsc.bitcast(d, jnp.int32), 1
                )
                xx = plsc.bitcast(u, jnp.float32)
                for _ in range(3):
                    xx = xx * (1.5 - 0.5 * d * xx * xx)
                deg_v[r, pl.ds(0, 16)] = xx

            pltpu.sync_copy(deg_v, dinv_hbm.at[cid].at[pl.ds(base, CHUNK)])
        plsc.subcore_barrier()

        def gather(j, b):
            off = pl.multiple_of(j * CHUNK, CHUNK)
            idx = src_v.at[pl.ds(off, CHUNK)]
            pltpu.async_copy(y_hbm.at[cid].at[idx], rows_i.at[b], gsems[b])
            pltpu.async_copy(dinv_hbm.at[cid].at[idx], dinv_v.at[b], dsems[b])

        def wait_gather(j, b):
            off = pl.multiple_of(j * CHUNK, CHUNK)
            idx = src_v.at[pl.ds(off, CHUNK)]
            pltpu.make_async_copy(y_hbm.at[cid].at[idx], rows_i.at[b], gsems[b]).wait()
            pltpu.make_async_copy(
                dinv_hbm.at[cid].at[idx], dinv_v.at[b], dsems[b]
            ).wait()

        def widen(b):
            ri = rows_i.at[b]
            rf = rows_f.at[b]
            dv = dinv_v.at[b]

            @pl.loop(0, CHUNK, unroll=8)
            def _(r):
                s = dv[r, pl.ds(0, 16)]
                for g in range(WPR):
                    u = ri[r, pl.ds(g * 16, 16)]
                    lo = plsc.bitcast(lax.shift_left(u, 16), jnp.float32) * s
                    hi = (
                        plsc.bitcast(lax.bitwise_and(u, jnp.int32(-65536)), jnp.float32)
                        * s
                    )
                    rf[r, pl.ds(g * 32, 16)] = lo
                    rf[r, pl.ds(g * 32 + 16, 16)] = hi

        def scatter(j, b):
            return pltpu.async_copy(
                rows_f.at[b], acc_sh.at[dst_v.at[j]], ssems[b], add=True
            )

        def wait_scatter(j, b):
            pltpu.make_async_copy(
                rows_f.at[b], acc_sh.at[dst_v.at[j]], ssems[b]
            ).wait()

        for b in range(NBUF):
            gather(b, b)

        @pl.loop(0, NCHUNK2, step=NBUF)
        def _(j0):
            for b in range(NBUF):
                j = j0 + b
                wait_gather(j, b)
                widen(b)
                scatter(j, b)
                jl = j - LAG
                bl = (b - LAG) % NBUF

                @pl.when(jl >= 0)
                def _():
                    wait_scatter(jl, bl)

                    @pl.when(jl + NBUF < NCHUNK2)
                    def _():
                        gather(jl + NBUF, bl)

        for t in range(LAG):
            j = NCHUNK2 - LAG + t
            wait_scatter(j, j % NBUF)
        plsc.subcore_barrier()
        pltpu.sync_copy(acc_sh.at[pl.ds(sid * STRIPE, STRIPE)], out_hbm.at[cid].at[pl.ds(sid * STRIPE, STRIPE)])

    return pl.kernel(
        body,
        out_type=[
            jax.ShapeDtypeStruct((NC, N_PAD, 16), jnp.float32),
            jax.ShapeDtypeStruct((NC, N_PAD, dh), jnp.float32),
        ],
        mesh=_mesh(),
        compiler_params=pltpu.CompilerParams(
            use_tc_tiling_on_sc=False, needs_layout_passes=False
        ),
        scratch_types=[
            pltpu.VMEM((E_TILE2,), jnp.int32),
            pltpu.VMEM((NCHUNK2, CHUNK), jnp.int32),
            pltpu.VMEM((CHUNK, 16), jnp.float32),
            pltpu.VMEM((CHUNK, 16), jnp.float32),
            pltpu.VMEM((NBUF, CHUNK, dh // 2), jnp.int32),
            pltpu.VMEM((NBUF, CHUNK, dh), jnp.float32),
            pltpu.VMEM((NBUF, CHUNK, 16), jnp.float32),
            pltpu.VMEM_SHARED((N_PAD, 16), jnp.float32),
            pltpu.VMEM_SHARED((N_PAD, dh), jnp.float32),
        ] + [pltpu.SemaphoreType.DMA] * (3 * NBUF + 1),
    )(y_bits, src1d, dst2d, ones_blk, zeros16, zeros_stripe)


def _sc_aggregate(y_bits, src1d, dst2d, zeros_stripe, dh):
    NBUF = 4
    LAG = 2
    WPR = dh // 32

    def body(y_hbm, src_hbm, dst_hbm, zeros_hbm, out_hbm, src_v, dst_v, rows_i, rows_f,
             acc_sh, *sems):
        gsems = sems[:NBUF]
        ssems = sems[NBUF:]
        cid = lax.axis_index("c")
        sid = lax.axis_index("s")
        pltpu.sync_copy(zeros_hbm, acc_sh.at[pl.ds(sid * STRIPE, STRIPE)])
        pltpu.sync_copy(src_hbm.at[pl.ds(sid * E_TILE2, E_TILE2)], src_v)
        pltpu.sync_copy(dst_hbm.at[pl.ds(sid * NCHUNK2, NCHUNK2)], dst_v)
        plsc.subcore_barrier()

        def gather(j, b):
            off = pl.multiple_of(j * CHUNK, CHUNK)
            return pltpu.async_copy(
                y_hbm.at[cid].at[src_v.at[pl.ds(off, CHUNK)]], rows_i.at[b], gsems[b]
            )

        def wait_gather(j, b):
            pltpu.make_async_copy(
                y_hbm.at[cid].at[
                    src_v.at[pl.ds(pl.multiple_of(j * CHUNK, CHUNK), CHUNK)]
                ],
                rows_i.at[b],
                gsems[b],
            ).wait()

        def widen(b):
            ri = rows_i.at[b]
            rf = rows_f.at[b]

            @pl.loop(0, CHUNK, unroll=8)
            def _(r):
                for g in range(WPR):
                    u = ri[r, pl.ds(g * 16, 16)]
                    lo = plsc.bitcast(lax.shift_left(u, 16), jnp.float32)
                    hi = plsc.bitcast(
                        lax.bitwise_and(u, jnp.int32(-65536)), jnp.float32
                    )
                    rf[r, pl.ds(g * 32, 16)] = lo
                    rf[r, pl.ds(g * 32 + 16, 16)] = hi

        def scatter(j, b):
            return pltpu.async_copy(
                rows_f.at[b], acc_sh.at[dst_v.at[j]], ssems[b], add=True
            )

        def wait_scatter(j, b):
            pltpu.make_async_copy(
                rows_f.at[b], acc_sh.at[dst_v.at[j]], ssems[b]
            ).wait()

        for b in range(NBUF):
            gather(b, b)

        @pl.loop(0, NCHUNK2, step=NBUF)
        def _(j0):
            for b in range(NBUF):
                j = j0 + b
                wait_gather(j, b)
                widen(b)
                scatter(j, b)
                jl = j - LAG
                bl = (b - LAG) % NBUF

                @pl.when(jl >= 0)
                def _():
                    wait_scatter(jl, bl)

                    @pl.when(jl + NBUF < NCHUNK2)
                    def _():
                        gather(jl + NBUF, bl)

        for t in range(LAG):
            j = NCHUNK2 - LAG + t
            wait_scatter(j, j % NBUF)
        plsc.subcore_barrier()
        pltpu.sync_copy(acc_sh.at[pl.ds(sid * STRIPE, STRIPE)], out_hbm.at[cid].at[pl.ds(sid * STRIPE, STRIPE)])

    return pl.kernel(
        body,
        out_type=jax.ShapeDtypeStruct((NC, N_PAD, dh), jnp.float32),
        mesh=_mesh(),
        compiler_params=pltpu.CompilerParams(
            use_tc_tiling_on_sc=False, needs_layout_passes=False
        ),
        scratch_types=[
            pltpu.VMEM((E_TILE2,), jnp.int32),
            pltpu.VMEM((NCHUNK2, CHUNK), jnp.int32),
            pltpu.VMEM((NBUF, CHUNK, dh // 2), jnp.int32),
            pltpu.VMEM((NBUF, CHUNK, dh), jnp.float32),
            pltpu.VMEM_SHARED((N_PAD, dh), jnp.float32),
        ] + [pltpu.SemaphoreType.DMA] * (2 * NBUF),
    )(y_bits, src1d, dst2d, zeros_stripe)


_ROW_BLK = 2000
_N_BLKS = N_NODES // _ROW_BLK


def _dinv_block(dinv_ref):
    return dinv_ref[0, :, 0:1]


def _tc_y1(x, w1, w1p):
    def body(x_ref, w_ref, wp_ref, y_ref, yl_ref):
        x = x_ref[...]
        y = jnp.dot(x, w_ref[...], preferred_element_type=jnp.float32)
        yb = y.astype(jnp.bfloat16)
        y_ref[0] = yb[:, : D_HID // 2]
        y_ref[1] = yb[:, D_HID // 2 :]
        yl = jnp.dot(x, wp_ref[...], preferred_element_type=jnp.float32)
        ylb = yl.astype(jnp.bfloat16)
        yl_ref[0] = ylb[:, : D_HID // 2]
        yl_ref[1] = ylb[:, D_HID // 2 :]

    return pl.pallas_call(
        body,
        grid=(_N_BLKS,),
        in_specs=[
            pl.BlockSpec((_ROW_BLK, D_IN), lambda i: (i, 0)),
            pl.BlockSpec((D_IN, D_HID), lambda i: (0, 0)),
            pl.BlockSpec((D_IN, D_HID), lambda i: (0, 0)),
        ],
        out_specs=[
            pl.BlockSpec((NC, _ROW_BLK, D_HID // 2), lambda i: (0, i, 0)),
            pl.BlockSpec((NC, _ROW_BLK, D_HID // 2), lambda i: (0, i, 0)),
        ],
        out_shape=[
            jax.ShapeDtypeStruct((NC, N_PAD, D_HID // 2), jnp.bfloat16),
            jax.ShapeDtypeStruct((NC, N_PAD, D_HID // 2), jnp.bfloat16),
        ],
    )(x, w1, w1p)


def _tc_h_y2(deg_parts, acc1, y1l, w2r, w2rp, b1l):
    def body(deg_ref, acc_ref, y1_ref, w_ref, wp_ref, b_ref, y2_ref, y2l_ref):
        dinv = _dinv_block(deg_ref)
        halves = []
        for c in range(NC):
            tot = acc_ref[c] + y1_ref[c].astype(jnp.float32) * dinv
            halves.append(jnp.maximum(tot * dinv + b_ref[c], 0.0))
        h = jnp.concatenate(halves, axis=1)
        y2 = jnp.dot(h, w_ref[...], preferred_element_type=jnp.float32) * dinv
        y2b = y2.astype(jnp.bfloat16)
        y2_ref[0] = y2b[:, : D_OUT // 2]
        y2_ref[1] = y2b[:, D_OUT // 2 :]
        y2l = jnp.dot(h, wp_ref[...], preferred_element_type=jnp.float32) * dinv
        y2lb = y2l.astype(jnp.bfloat16)
        y2l_ref[0] = y2lb[:, : D_OUT // 2]
        y2l_ref[1] = y2lb[:, D_OUT // 2 :]

    return pl.pallas_call(
        body,
        grid=(_N_BLKS,),
        in_specs=[
            pl.BlockSpec((NC, _ROW_BLK, 16), lambda i: (0, i, 0)),
            pl.BlockSpec((NC, _ROW_BLK, D_HID // 2), lambda i: (0, i, 0)),
            pl.BlockSpec((NC, _ROW_BLK, D_HID // 2), lambda i: (0, i, 0)),
            pl.BlockSpec((D_HID, D_OUT), lambda i: (0, 0)),
            pl.BlockSpec((D_HID, D_OUT), lambda i: (0, 0)),
            pl.BlockSpec((NC, 1, D_HID // 2), lambda i: (0, 0, 0)),
        ],
        out_specs=[
            pl.BlockSpec((NC, _ROW_BLK, D_OUT // 2), lambda i: (0, i, 0)),
            pl.BlockSpec((NC, _ROW_BLK, D_OUT // 2), lambda i: (0, i, 0)),
        ],
        out_shape=[
            jax.ShapeDtypeStruct((NC, N_PAD, D_OUT // 2), jnp.bfloat16),
            jax.ShapeDtypeStruct((NC, N_PAD, D_OUT // 2), jnp.bfloat16),
        ],
    )(deg_parts, acc1, y1l, w2r, w2rp, b1l)


def _tc_out(deg_parts, acc2, y2, b2):
    def body(deg_ref, acc_ref, y2_ref, b_ref, o_ref):
        dinv = _dinv_block(deg_ref)
        halves = []
        for c in range(NC):
            tot = acc_ref[c] + y2_ref[c].astype(jnp.float32)
            halves.append(jnp.maximum(tot * dinv + b_ref[c], 0.0))
        o_ref[...] = jnp.concatenate(halves, axis=1)

    return pl.pallas_call(
        body,
        grid=(_N_BLKS,),
        in_specs=[
            pl.BlockSpec((NC, _ROW_BLK, 16), lambda i: (0, i, 0)),
            pl.BlockSpec((NC, _ROW_BLK, D_OUT // 2), lambda i: (0, i, 0)),
            pl.BlockSpec((NC, _ROW_BLK, D_OUT // 2), lambda i: (0, i, 0)),
            pl.BlockSpec((NC, 1, D_OUT // 2), lambda i: (0, 0, 0)),
        ],
        out_specs=pl.BlockSpec((_ROW_BLK, D_OUT), lambda i: (i, 0)),
        out_shape=jax.ShapeDtypeStruct((N_NODES, D_OUT), jnp.float32),
    )(deg_parts, acc2, y2, b2)


def _widen_perm(d):
    dh = d // 2
    perm = []
    for c in range(NC):
        for g in range(dh // 32):
            perm += [c * dh + 32 * g + 2 * k for k in range(16)]
            perm += [c * dh + 32 * g + 2 * k + 1 for k in range(16)]
    return perm


def _pack_bits(y_bf):
    nc, n, dh = y_bf.shape
    return jax.lax.bitcast_convert_type(
        y_bf.reshape(nc, n, dh // 2, 2), jnp.int32
    )


def kernel(x, edge_index, W1, b1, W2, b2):
    src = edge_index[0].astype(jnp.int32)
    dst = edge_index[1].astype(jnp.int32)
    pad = jnp.full((E_PAD - E,), N_NODES, jnp.int32)
    src_pad = jnp.concatenate([src, pad])
    dst2d = jnp.concatenate([dst, pad]).reshape(E_PAD // CHUNK, CHUNK)
    ones16 = jnp.ones((CHUNK, 16), jnp.float32)
    z16 = jnp.zeros((STRIPE, 16), jnp.float32)
    z64 = jnp.zeros((STRIPE, D_HID // 2), jnp.float32)
    z32 = jnp.zeros((STRIPE, D_OUT // 2), jnp.float32)

    perm1 = _widen_perm(D_HID)
    perm2 = _widen_perm(D_OUT)
    inv2 = [0] * D_OUT
    for j, o in enumerate(perm2):
        inv2[o] = j
    p1 = jnp.asarray(perm1, jnp.int32)
    p2 = jnp.asarray(perm2, jnp.int32)
    w1p = W1[:, p1]
    b1l = b1[p1].reshape(NC, 1, -1)
    w2r = W2[p1, :]
    w2rp = w2r[:, p2]
    b2l = b2[p2].reshape(NC, 1, -1)

    y1, y1l = _tc_y1(x, W1, w1p)
    dinv16, acc1 = _sc_deg_agg1(
        _pack_bits(y1), src_pad, dst2d, ones16, z16, z64, D_HID // 2
    )
    y2, y2l = _tc_h_y2(dinv16, acc1, y1l, w2r, w2rp, b1l)
    acc2 = _sc_aggregate(_pack_bits(y2), src_pad, dst2d, z32, D_OUT // 2)
    out = _tc_out(dinv16, acc2, y2l, b2l)
    return out[:, jnp.asarray(inv2, jnp.int32)]


# speedup vs baseline: 1.4282x; 1.4282x over previous
import functools

import jax
import jax.numpy as jnp
from jax import lax
from jax.experimental import pallas as pl
from jax.experimental.pallas import tpu as pltpu
from jax.experimental.pallas import tpu_sc as plsc

N_NODES = 10000
D_IN = 128
D_HID = 128
D_OUT = 64
N_PAD = 10240
E = 320000
CHUNK = 128
NC, NS = 2, 16
NW = NC * NS
E_TILE = 10240
E_PAD = E_TILE * NW
NCHUNK = E_TILE // CHUNK
STRIPE = N_PAD // NS


def _mesh():
    return plsc.VectorSubcoreMesh(
        core_axis_name="c", subcore_axis_name="s", num_cores=NC, num_subcores=NS
    )


def _sc_degree(dst2d, ones_blk, zeros_stripe):
    def body(dst_hbm, ones_hbm, zeros_hbm, out_hbm, dst_v, ones_v, acc_sh):
        cid = lax.axis_index("c")
        sid = lax.axis_index("s")
        wid = cid * NS + sid
        pltpu.sync_copy(zeros_hbm, acc_sh.at[pl.ds(sid * STRIPE, STRIPE)])
        pltpu.sync_copy(ones_hbm, ones_v)
        pltpu.sync_copy(dst_hbm.at[pl.ds(wid * NCHUNK, NCHUNK)], dst_v)
        plsc.subcore_barrier()

        def step(j, carry):
            pltpu.sync_copy(ones_v, acc_sh.at[dst_v.at[j]], add=True)
            return carry

        lax.fori_loop(0, NCHUNK, step, 0)
        plsc.subcore_barrier()
        pltpu.sync_copy(
            acc_sh.at[pl.ds(sid * STRIPE, STRIPE)],
            out_hbm.at[cid].at[pl.ds(sid * STRIPE, STRIPE)],
        )

    return pl.kernel(
        body,
        out_type=jax.ShapeDtypeStruct((NC, N_PAD, 8), jnp.float32),
        mesh=_mesh(),
        compiler_params=pltpu.CompilerParams(use_tc_tiling_on_sc=False),
        scratch_types=[
            pltpu.VMEM((NCHUNK, CHUNK), jnp.int32),
            pltpu.VMEM((CHUNK, 8), jnp.float32),
            pltpu.VMEM_SHARED((N_PAD, 8), jnp.float32),
        ],
    )(dst2d, ones_blk, zeros_stripe)


E_TILE2 = E_PAD // NS
NCHUNK2 = E_TILE2 // CHUNK


def _sc_aggregate(y_bits, src1d, dst2d, zeros_stripe, dh):
    NBUF = 4
    LAG = 2
    WPR = dh // 32

    def body(y_hbm, src_hbm, dst_hbm, zeros_hbm, out_hbm, src_v, dst_v, rows_i, rows_f,
             acc_sh, *sems):
        gsems = sems[:NBUF]
        ssems = sems[NBUF:]
        cid = lax.axis_index("c")
        sid = lax.axis_index("s")
        pltpu.sync_copy(zeros_hbm, acc_sh.at[pl.ds(sid * STRIPE, STRIPE)])
        pltpu.sync_copy(src_hbm.at[pl.ds(sid * E_TILE2, E_TILE2)], src_v)
        pltpu.sync_copy(dst_hbm.at[pl.ds(sid * NCHUNK2, NCHUNK2)], dst_v)
        plsc.subcore_barrier()

        def gather(j, b):
            off = pl.multiple_of(j * CHUNK, CHUNK)
            return pltpu.async_copy(
                y_hbm.at[cid].at[src_v.at[pl.ds(off, CHUNK)]], rows_i.at[b], gsems[b]
            )

        def wait_gather(j, b):
            pltpu.make_async_copy(
                y_hbm.at[cid].at[
                    src_v.at[pl.ds(pl.multiple_of(j * CHUNK, CHUNK), CHUNK)]
                ],
                rows_i.at[b],
                gsems[b],
            ).wait()

        def widen(b):
            ri = rows_i.at[b]
            rf = rows_f.at[b]

            @pl.loop(0, CHUNK, unroll=8)
            def _(r):
                for g in range(WPR):
                    u = ri[r, pl.ds(g * 16, 16)]
                    lo = plsc.bitcast(lax.shift_left(u, 16), jnp.float32)
                    hi = plsc.bitcast(
                        lax.bitwise_and(u, jnp.int32(-65536)), jnp.float32
                    )
                    rf[r, pl.ds(g * 32, 16)] = lo
                    rf[r, pl.ds(g * 32 + 16, 16)] = hi

        def scatter(j, b):
            return pltpu.async_copy(
                rows_f.at[b], acc_sh.at[dst_v.at[j]], ssems[b], add=True
            )

        def wait_scatter(j, b):
            pltpu.make_async_copy(
                rows_f.at[b], acc_sh.at[dst_v.at[j]], ssems[b]
            ).wait()

        for b in range(NBUF):
            gather(b, b)

        @pl.loop(0, NCHUNK2, step=NBUF)
        def _(j0):
            for b in range(NBUF):
                j = j0 + b
                wait_gather(j, b)
                widen(b)
                scatter(j, b)
                jl = j - LAG
                bl = (b - LAG) % NBUF

                @pl.when(jl >= 0)
                def _():
                    wait_scatter(jl, bl)

                    @pl.when(jl + NBUF < NCHUNK2)
                    def _():
                        gather(jl + NBUF, bl)

        for t in range(LAG):
            j = NCHUNK2 - LAG + t
            wait_scatter(j, j % NBUF)
        plsc.subcore_barrier()
        pltpu.sync_copy(acc_sh.at[pl.ds(sid * STRIPE, STRIPE)], out_hbm.at[cid].at[pl.ds(sid * STRIPE, STRIPE)])

    return pl.kernel(
        body,
        out_type=jax.ShapeDtypeStruct((NC, N_PAD, dh), jnp.float32),
        mesh=_mesh(),
        compiler_params=pltpu.CompilerParams(
            use_tc_tiling_on_sc=False, needs_layout_passes=False
        ),
        scratch_types=[
            pltpu.VMEM((E_TILE2,), jnp.int32),
            pltpu.VMEM((NCHUNK2, CHUNK), jnp.int32),
            pltpu.VMEM((NBUF, CHUNK, dh // 2), jnp.int32),
            pltpu.VMEM((NBUF, CHUNK, dh), jnp.float32),
            pltpu.VMEM_SHARED((N_PAD, dh), jnp.float32),
        ] + [pltpu.SemaphoreType.DMA] * (2 * NBUF),
    )(y_bits, src1d, dst2d, zeros_stripe)


_ROW_BLK = 2000
_N_BLKS = N_NODES // _ROW_BLK


def _dinv_block(deg_ref):
    deg = deg_ref[0, :, 0:1] + deg_ref[1, :, 0:1] + 1.0
    return lax.rsqrt(deg)


def _tc_y1(deg_parts, x_pad, w1, w1p):
    def body(deg_ref, x_ref, w_ref, wp_ref, y_ref, yl_ref):
        dinv = _dinv_block(deg_ref)
        x = x_ref[...]
        y = jnp.dot(x, w_ref[...], preferred_element_type=jnp.float32) * dinv
        yb = y.astype(jnp.bfloat16)
        y_ref[0] = yb[:, : D_HID // 2]
        y_ref[1] = yb[:, D_HID // 2 :]
        yl = jnp.dot(x, wp_ref[...], preferred_element_type=jnp.float32) * dinv
        ylb = yl.astype(jnp.bfloat16)
        yl_ref[0] = ylb[:, : D_HID // 2]
        yl_ref[1] = ylb[:, D_HID // 2 :]

    return pl.pallas_call(
        body,
        grid=(_N_BLKS,),
        in_specs=[
            pl.BlockSpec((NC, _ROW_BLK, 8), lambda i: (0, i, 0)),
            pl.BlockSpec((_ROW_BLK, D_IN), lambda i: (i, 0)),
            pl.BlockSpec((D_IN, D_HID), lambda i: (0, 0)),
            pl.BlockSpec((D_IN, D_HID), lambda i: (0, 0)),
        ],
        out_specs=[
            pl.BlockSpec((NC, _ROW_BLK, D_HID // 2), lambda i: (0, i, 0)),
            pl.BlockSpec((NC, _ROW_BLK, D_HID // 2), lambda i: (0, i, 0)),
        ],
        out_shape=[
            jax.ShapeDtypeStruct((NC, N_PAD, D_HID // 2), jnp.bfloat16),
            jax.ShapeDtypeStruct((NC, N_PAD, D_HID // 2), jnp.bfloat16),
        ],
    )(deg_parts, x_pad, w1, w1p)


def _tc_h_y2(deg_parts, acc1, y1l, w2r, w2rp, b1l):
    def body(deg_ref, acc_ref, y1_ref, w_ref, wp_ref, b_ref, y2_ref, y2l_ref):
        dinv = _dinv_block(deg_ref)
        halves = []
        for c in range(NC):
            tot = acc_ref[c] + y1_ref[c].astype(jnp.float32)
            halves.append(jnp.maximum(tot * dinv + b_ref[c], 0.0))
        h = jnp.concatenate(halves, axis=1)
        y2 = jnp.dot(h, w_ref[...], preferred_element_type=jnp.float32) * dinv
        y2b = y2.astype(jnp.bfloat16)
        y2_ref[0] = y2b[:, : D_OUT // 2]
        y2_ref[1] = y2b[:, D_OUT // 2 :]
        y2l = jnp.dot(h, wp_ref[...], preferred_element_type=jnp.float32) * dinv
        y2lb = y2l.astype(jnp.bfloat16)
        y2l_ref[0] = y2lb[:, : D_OUT // 2]
        y2l_ref[1] = y2lb[:, D_OUT // 2 :]

    return pl.pallas_call(
        body,
        grid=(_N_BLKS,),
        in_specs=[
            pl.BlockSpec((NC, _ROW_BLK, 8), lambda i: (0, i, 0)),
            pl.BlockSpec((NC, _ROW_BLK, D_HID // 2), lambda i: (0, i, 0)),
            pl.BlockSpec((NC, _ROW_BLK, D_HID // 2), lambda i: (0, i, 0)),
            pl.BlockSpec((D_HID, D_OUT), lambda i: (0, 0)),
            pl.BlockSpec((D_HID, D_OUT), lambda i: (0, 0)),
            pl.BlockSpec((NC, 1, D_HID // 2), lambda i: (0, 0, 0)),
        ],
        out_specs=[
            pl.BlockSpec((NC, _ROW_BLK, D_OUT // 2), lambda i: (0, i, 0)),
            pl.BlockSpec((NC, _ROW_BLK, D_OUT // 2), lambda i: (0, i, 0)),
        ],
        out_shape=[
            jax.ShapeDtypeStruct((NC, N_PAD, D_OUT // 2), jnp.bfloat16),
            jax.ShapeDtypeStruct((NC, N_PAD, D_OUT // 2), jnp.bfloat16),
        ],
    )(deg_parts, acc1, y1l, w2r, w2rp, b1l)


def _tc_out(deg_parts, acc2, y2, b2):
    def body(deg_ref, acc_ref, y2_ref, b_ref, o_ref):
        dinv = _dinv_block(deg_ref)
        halves = []
        for c in range(NC):
            tot = acc_ref[c] + y2_ref[c].astype(jnp.float32)
            halves.append(jnp.maximum(tot * dinv + b_ref[c], 0.0))
        o_ref[...] = jnp.concatenate(halves, axis=1)

    return pl.pallas_call(
        body,
        grid=(_N_BLKS,),
        in_specs=[
            pl.BlockSpec((NC, _ROW_BLK, 8), lambda i: (0, i, 0)),
            pl.BlockSpec((NC, _ROW_BLK, D_OUT // 2), lambda i: (0, i, 0)),
            pl.BlockSpec((NC, _ROW_BLK, D_OUT // 2), lambda i: (0, i, 0)),
            pl.BlockSpec((NC, 1, D_OUT // 2), lambda i: (0, 0, 0)),
        ],
        out_specs=pl.BlockSpec((_ROW_BLK, D_OUT), lambda i: (i, 0)),
        out_shape=jax.ShapeDtypeStruct((N_NODES, D_OUT), jnp.float32),
    )(deg_parts, acc2, y2, b2)


def _widen_perm(d):
    dh = d // 2
    perm = []
    for c in range(NC):
        for g in range(dh // 32):
            perm += [c * dh + 32 * g + 2 * k for k in range(16)]
            perm += [c * dh + 32 * g + 2 * k + 1 for k in range(16)]
    return perm


def _pack_bits(y_bf):
    nc, n, dh = y_bf.shape
    return jax.lax.bitcast_convert_type(
        y_bf.reshape(nc, n, dh // 2, 2), jnp.int32
    )


def kernel(x, edge_index, W1, b1, W2, b2):
    src = edge_index[0].astype(jnp.int32)
    dst = edge_index[1].astype(jnp.int32)
    pad = jnp.full((E_PAD - E,), N_NODES, jnp.int32)
    src_pad = jnp.concatenate([src, pad])
    dst2d = jnp.concatenate([dst, pad]).reshape(E_PAD // CHUNK, CHUNK)
    ones16 = jnp.ones((CHUNK, 8), jnp.float32)
    z16 = jnp.zeros((STRIPE, 8), jnp.float32)
    z64 = jnp.zeros((STRIPE, D_HID // 2), jnp.float32)
    z32 = jnp.zeros((STRIPE, D_OUT // 2), jnp.float32)

    perm1 = _widen_perm(D_HID)
    perm2 = _widen_perm(D_OUT)
    inv2 = [0] * D_OUT
    for j, o in enumerate(perm2):
        inv2[o] = j
    p1 = jnp.asarray(perm1, jnp.int32)
    p2 = jnp.asarray(perm2, jnp.int32)
    w1p = W1[:, p1]
    b1l = b1[p1].reshape(NC, 1, -1)
    w2r = W2[p1, :]
    w2rp = w2r[:, p2]
    b2l = b2[p2].reshape(NC, 1, -1)

    deg_parts = _sc_degree(dst2d, ones16, z16)
    y1, y1l = _tc_y1(deg_parts, x, W1, w1p)
    acc1 = _sc_aggregate(_pack_bits(y1), src_pad, dst2d, z64, D_HID // 2)
    y2, y2l = _tc_h_y2(deg_parts, acc1, y1l, w2r, w2rp, b1l)
    acc2 = _sc_aggregate(_pack_bits(y2), src_pad, dst2d, z32, D_OUT // 2)
    out = _tc_out(deg_parts, acc2, y2l, b2l)
    return out[:, jnp.asarray(inv2, jnp.int32)]
